# baseline (speedup 1.0000x reference)
import jax
import jax.numpy as jnp
from jax.experimental import pallas as pl
from jax.experimental.pallas import tpu as pltpu

_D = 512
_P = 8
_T = 128
_BB = 8


def _body(x_ref, w_ref, b_ref, pos_ref, o_ref):
    x = x_ref[...]
    y = jax.lax.dot_general(
        x.reshape(_BB * _T, _P), w_ref[...],
        (((1,), (0,)), ((), ())),
        preferred_element_type=jnp.float32,
    )
    add = pos_ref[...] + b_ref[...]
    o_ref[...] = y.reshape(_BB, _T, _D) + add[None]


def kernel(inputs, W_proj, b_proj, pos_table):
    B = inputs.shape[0]
    x3 = inputs.reshape(B, _T, _P)
    b2 = b_proj.reshape(1, _D)
    return pl.pallas_call(
        _body,
        grid=(B // _BB,),
        in_specs=[
            pl.BlockSpec((_BB, _T, _P), lambda i: (i, 0, 0)),
            pl.BlockSpec((_P, _D), lambda i: (0, 0)),
            pl.BlockSpec((1, _D), lambda i: (0, 0)),
            pl.BlockSpec((_T, _D), lambda i: (0, 0)),
        ],
        out_specs=pl.BlockSpec((_BB, _T, _D), lambda i: (i, 0, 0)),
        out_shape=jax.ShapeDtypeStruct((B, _T, _D), jnp.float32),
        compiler_params=pltpu.CompilerParams(
            dimension_semantics=("arbitrary",),
        ),
    )(x3, W_proj, b2, pos_table)
